# Initial kernel scaffold; baseline (speedup 1.0000x reference)
import jax
import jax.numpy as jnp
from jax.experimental import pallas as pl


def kernel(key_soft, W):
    raise NotImplementedError("write your pallas kernel here")


# speedup vs baseline: 8.2341x; 8.2341x over previous
import jax
import jax.numpy as jnp
from jax.experimental import pallas as pl

_B = 16
_T = 576
_E = 64
_N = 1025
_NE = 1024
_LEGACY = 0.2
_TILE = 512


def _c1_body(ks_ref, wt_ref, adv_ref, minidx_ref):
    ks = ks_ref[...]
    wt = wt_ref[...]
    rowssq = jnp.sum(ks * ks, axis=1, keepdims=True)
    wsq = jnp.sum(wt * wt, axis=0)
    mm = jax.lax.dot_general(ks, wt, (((1,), (0,)), ((), ())),
                             preferred_element_type=jnp.float32)
    d = rowssq + wsq[None, :] - 2.0 * mm
    dmin = jnp.min(d, axis=1, keepdims=True)
    ii = jax.lax.broadcasted_iota(jnp.int32, d.shape, 1)
    minidx = jnp.min(jnp.where(d == dmin, ii, jnp.int32(2**30)), axis=1)
    minidx_ref[...] = minidx[:, None]
    advb = (d[:, :_NE] > d[:, 1:_N]) & (ii[:, :_NE] < (_NE - 1))
    adv_ref[...] = advb.astype(jnp.bfloat16)


def _call1(ksf, wt):
    n_tiles = ksf.shape[0] // _TILE
    return pl.pallas_call(
        _c1_body,
        grid=(n_tiles,),
        in_specs=[pl.BlockSpec((_TILE, _E), lambda i: (i, 0)),
                  pl.BlockSpec((_E, _N), lambda i: (0, 0))],
        out_specs=[pl.BlockSpec((_TILE, _NE), lambda i: (i, 0)),
                   pl.BlockSpec((_TILE, 1), lambda i: (i, 0))],
        out_shape=[jax.ShapeDtypeStruct((_T * _B, _NE), jnp.bfloat16),
                   jax.ShapeDtypeStruct((_T * _B, 1), jnp.int32)],
    )(ksf, wt)


def _c2_body(adv_ref, mi_ref, enc_ref):
    enc0 = jnp.minimum(mi_ref[...][:, 0], _NE - 1)
    lane = jax.lax.broadcasted_iota(jnp.int32, (_B, _NE), 1)
    p0 = (lane == enc0[:, None]).astype(jnp.float32)
    enc_ref[pl.ds(0, _B), :] = enc0[:, None]

    def step(t, carry):
        p, ind = carry
        a = adv_ref[pl.ds(t * _B, _B), :].astype(jnp.float32)
        q = p * a
        ind = ind + jnp.sum(q, axis=1)
        p = p - q + jnp.concatenate(
            [jnp.zeros((_B, 1), jnp.float32), q[:, :_NE - 1]], axis=1)
        enc_ref[pl.ds(t * _B, _B), :] = ind.astype(jnp.int32)[:, None]
        return p, ind

    jax.lax.fori_loop(1, _T, step, (p0, enc0.astype(jnp.float32)))


def _call2(adv, minidx):
    return pl.pallas_call(
        _c2_body,
        grid=(1,),
        in_specs=[pl.BlockSpec((_T * _B, _NE), lambda i: (0, 0)),
                  pl.BlockSpec((_B, 1), lambda i: (0, 0))],
        out_specs=pl.BlockSpec((_T * _B, 1), lambda i: (0, 0)),
        out_shape=jax.ShapeDtypeStruct((_T * _B, 1), jnp.int32),
    )(adv, minidx)


def _c3_body(ks_ref, w_ref, ind_ref, mi_ref,
             kh_ref, lh_ref, ln_ref, en_ref):
    ks = ks_ref[...]
    w = w_ref[...]
    ind = ind_ref[...]
    indn = jnp.minimum(ind + 1, _NE - 1)
    mi = mi_ref[...]
    jj = jax.lax.broadcasted_iota(jnp.int32, (_TILE, _N), 1)
    dn = (((1,), (0,)), ((), ()))
    khh = jax.lax.dot_general((jj == ind).astype(jnp.float32), w, dn,
                              preferred_element_type=jnp.float32)
    khn = jax.lax.dot_general((jj == indn).astype(jnp.float32), w, dn,
                              preferred_element_type=jnp.float32)
    kmin = jax.lax.dot_general((jj == mi).astype(jnp.float32), w, dn,
                               preferred_element_type=jnp.float32)
    dh = ks - khh
    s_here = jnp.sum(dh * dh, axis=1)
    dnx = ks - khn
    s_next = jnp.sum(dnx * dnx, axis=1)
    dm = ks - kmin
    s_min = jnp.sum(dm * dm, axis=1)
    base_h = s_here + s_here * _LEGACY
    base_n = s_next + s_next * _LEGACY
    lmi = s_min + s_min * _LEGACY
    lm_h = jnp.where(lmi < base_h, lmi, 0.0)
    lm_n = jnp.where(lmi < base_n, lmi, 0.0)
    dd = s_next - s_here
    kh_ref[...] = ks + (khh - ks)
    lh_ref[...] = (base_h + (-base_n) - lm_h)[:, None]
    ln_ref[...] = (base_n + (-base_h) - lm_n)[:, None]
    en_ref[...] = (dd + dd * _LEGACY)[:, None]


def _call3(ksf, w, enc, minidx):
    n_tiles = ksf.shape[0] // _TILE
    return pl.pallas_call(
        _c3_body,
        grid=(n_tiles,),
        in_specs=[pl.BlockSpec((_TILE, _E), lambda i: (i, 0)),
                  pl.BlockSpec((_N, _E), lambda i: (0, 0)),
                  pl.BlockSpec((_TILE, 1), lambda i: (i, 0)),
                  pl.BlockSpec((_TILE, 1), lambda i: (i, 0))],
        out_specs=[pl.BlockSpec((_TILE, _E), lambda i: (i, 0)),
                   pl.BlockSpec((_TILE, 1), lambda i: (i, 0)),
                   pl.BlockSpec((_TILE, 1), lambda i: (i, 0)),
                   pl.BlockSpec((_TILE, 1), lambda i: (i, 0))],
        out_shape=[jax.ShapeDtypeStruct((_T * _B, _E), jnp.float32),
                   jax.ShapeDtypeStruct((_T * _B, 1), jnp.float32),
                   jax.ShapeDtypeStruct((_T * _B, 1), jnp.float32),
                   jax.ShapeDtypeStruct((_T * _B, 1), jnp.float32)],
    )(ksf, w, enc, minidx)


def _c4_body(en_ref, enc_ref, v_ref, em_ref, led_ref):
    en = en_ref[...]
    enc = enc_ref[...]
    change = (enc[1:, :] - enc[:-1, :]) != 0
    ec = jnp.where(change, 0.0, en[1:, :] - en[:-1, :])
    led = jnp.mean(jnp.maximum(ec + (1e-06 / _NE), 0.0))
    mn = jnp.min(enc, axis=0)
    mx = jnp.max(enc, axis=0)
    v_ref[...] = jnp.reshape(jnp.max(mx - mn), (1, 1))
    em_ref[...] = jnp.reshape(jnp.mean(en), (1, 1))
    led_ref[...] = jnp.reshape(led, (1, 1))


def _call4(en_tb, enc_tb):
    return pl.pallas_call(
        _c4_body,
        grid=(1,),
        in_specs=[pl.BlockSpec((_T, _B), lambda i: (0, 0)),
                  pl.BlockSpec((_T, _B), lambda i: (0, 0))],
        out_specs=[pl.BlockSpec((1, 1), lambda i: (0, 0)),
                   pl.BlockSpec((1, 1), lambda i: (0, 0)),
                   pl.BlockSpec((1, 1), lambda i: (0, 0))],
        out_shape=[jax.ShapeDtypeStruct((1, 1), jnp.int32),
                   jax.ShapeDtypeStruct((1, 1), jnp.float32),
                   jax.ShapeDtypeStruct((1, 1), jnp.float32)],
    )(en_tb, enc_tb)


def kernel(key_soft, W):
    B, T, E = key_soft.shape
    ksf = key_soft.transpose(1, 0, 2).reshape(T * B, E)
    wt = W.T
    adv, minidx = _call1(ksf, wt)
    enc = _call2(adv, minidx)
    kh, lh, ln, en = _call3(ksf, W, enc, minidx)
    v, em, led = _call4(en.reshape(T, B), enc.reshape(T, B))
    key_hard = kh.reshape(T, B, E).transpose(1, 0, 2)
    encoding_indices = enc.reshape(T, B).T
    loss_here = lh.reshape(T, B).T
    loss_next = ln.reshape(T, B).T
    return (key_hard, encoding_indices, v[0, 0], loss_here, loss_next,
            em[0, 0], led[0, 0])


# speedup vs baseline: 9.9145x; 1.2041x over previous
import dataclasses

import jax
import jax.numpy as jnp
from jax.experimental import pallas as pl
from jax.experimental.pallas import tpu as pltpu
from jax.experimental.pallas import tpu_sc as plsc

_B = 16
_T = 576
_E = 64
_N = 1025
_NE = 1024
_LEGACY = 0.2
_TILE = 512


def _c1_body(ks_ref, wt_ref, w32_ref, minidx_ref):
    ks = ks_ref[...]
    wt = wt_ref[...]
    rowssq = jnp.sum(ks * ks, axis=1, keepdims=True)
    wsq = jnp.sum(wt * wt, axis=0)
    mm = jax.lax.dot_general(ks, wt, (((1,), (0,)), ((), ())),
                             preferred_element_type=jnp.float32)
    d = rowssq + wsq[None, :] - 2.0 * mm
    dmin = jnp.min(d, axis=1, keepdims=True)
    ii = jax.lax.broadcasted_iota(jnp.int32, d.shape, 1)
    minidx = jnp.min(jnp.where(d == dmin, ii, jnp.int32(2**30)), axis=1)
    minidx_ref[...] = minidx[:, None]
    advb = (d[:, :_NE] > d[:, 1:_N]) & (ii[:, :_NE] < (_NE - 1))
    advf = advb.astype(jnp.float32)
    jrow = jax.lax.broadcasted_iota(jnp.int32, (_NE, 32), 0)
    kcol = jax.lax.broadcasted_iota(jnp.int32, (_NE, 32), 1)
    bitpos = jrow & 31
    hit = (jrow >> 5) == kcol
    in_lo = bitpos < 16
    p_lo = jnp.where(hit & in_lo,
                     (1 << jnp.where(in_lo, bitpos, 0)).astype(jnp.float32),
                     0.0)
    p_hi = jnp.where(hit & (~in_lo),
                     (1 << jnp.maximum(bitpos - 16, 0)).astype(jnp.float32),
                     0.0)
    dn = (((1,), (0,)), ((), ()))
    lo = jax.lax.dot_general(advf, p_lo, dn,
                             preferred_element_type=jnp.float32)
    hi = jax.lax.dot_general(advf, p_hi, dn,
                             preferred_element_type=jnp.float32)
    w32_ref[...] = lo.astype(jnp.int32) | (hi.astype(jnp.int32) << 16)


def _call1(ksf, wt):
    n_tiles = ksf.shape[0] // _TILE
    return pl.pallas_call(
        _c1_body,
        grid=(n_tiles,),
        in_specs=[pl.BlockSpec((_TILE, _E), lambda i: (i, 0)),
                  pl.BlockSpec((_E, _N), lambda i: (0, 0))],
        out_specs=[pl.BlockSpec((_TILE, 32), lambda i: (i, 0)),
                   pl.BlockSpec((_TILE, 1), lambda i: (i, 0))],
        out_shape=[jax.ShapeDtypeStruct((_T * _B, 32), jnp.int32),
                   jax.ShapeDtypeStruct((_T * _B, 1), jnp.int32)],
    )(ksf, wt)


def _call2_sc(w32_bm, mi16):
    mesh = plsc.VectorSubcoreMesh(core_axis_name="c", subcore_axis_name="s",
                                  num_cores=2, num_subcores=16)
    cp = pltpu.CompilerParams()
    if "needs_layout_passes" in pltpu.CompilerParams.__dataclass_fields__:
        cp = dataclasses.replace(cp, needs_layout_passes=False)

    @pl.kernel(
        out_type=jax.ShapeDtypeStruct((_B, _T * 16), jnp.int32),
        mesh=mesh,
        compiler_params=cp,
        scratch_types=[pltpu.VMEM((_T * 32,), jnp.int32),
                       pltpu.VMEM((_T * 16,), jnp.int32),
                       pltpu.VMEM((_B,), jnp.int32),
                       pltpu.SemaphoreType.DMA],
    )
    def scan_kernel(w32_ref, mi_ref, enc_ref, words, encv, miv, sem):
        c = jax.lax.axis_index("c")
        s = jax.lax.axis_index("s")
        b = c * (_B // 2) + s

        @pl.when(s < (_B // 2))
        def _():
            pltpu.async_copy(w32_ref.at[b], words, sem).wait()
            pltpu.async_copy(mi_ref, miv, sem).wait()
            bvec = jnp.zeros((16,), jnp.int32) + b
            ind0 = jnp.minimum(plsc.load_gather(miv, [bvec]), _NE - 1)
            encv[pl.ds(0, 16)] = ind0

            def step(t, ind):
                w = plsc.load_gather(words, [t * 32 + (ind >> 5)])
                bit = (w >> (ind & 31)) & 1
                ind = ind + bit
                encv[pl.ds(t * 16, 16)] = ind
                return ind

            jax.lax.fori_loop(1, _T, step, ind0)
            pltpu.async_copy(encv, enc_ref.at[b], sem).wait()

    return scan_kernel(w32_bm, mi16)


def _c3_body(ks_ref, w_ref, ind_ref, mi_ref,
             kh_ref, lh_ref, ln_ref, en_ref):
    ks = ks_ref[...]
    w = w_ref[...]
    ind = ind_ref[...]
    indn = jnp.minimum(ind + 1, _NE - 1)
    mi = mi_ref[...]
    jj = jax.lax.broadcasted_iota(jnp.int32, (_TILE, _N), 1)
    dn = (((1,), (0,)), ((), ()))
    khh = jax.lax.dot_general((jj == ind).astype(jnp.float32), w, dn,
                              preferred_element_type=jnp.float32)
    khn = jax.lax.dot_general((jj == indn).astype(jnp.float32), w, dn,
                              preferred_element_type=jnp.float32)
    kmin = jax.lax.dot_general((jj == mi).astype(jnp.float32), w, dn,
                               preferred_element_type=jnp.float32)
    dh = ks - khh
    s_here = jnp.sum(dh * dh, axis=1)
    dnx = ks - khn
    s_next = jnp.sum(dnx * dnx, axis=1)
    dm = ks - kmin
    s_min = jnp.sum(dm * dm, axis=1)
    base_h = s_here + s_here * _LEGACY
    base_n = s_next + s_next * _LEGACY
    lmi = s_min + s_min * _LEGACY
    lm_h = jnp.where(lmi < base_h, lmi, 0.0)
    lm_n = jnp.where(lmi < base_n, lmi, 0.0)
    dd = s_next - s_here
    kh_ref[...] = ks + (khh - ks)
    lh_ref[...] = (base_h + (-base_n) - lm_h)[:, None]
    ln_ref[...] = (base_n + (-base_h) - lm_n)[:, None]
    en_ref[...] = (dd + dd * _LEGACY)[:, None]


def _call3(ksf, w, enc, minidx):
    n_tiles = ksf.shape[0] // _TILE
    return pl.pallas_call(
        _c3_body,
        grid=(n_tiles,),
        in_specs=[pl.BlockSpec((_TILE, _E), lambda i: (i, 0)),
                  pl.BlockSpec((_N, _E), lambda i: (0, 0)),
                  pl.BlockSpec((_TILE, 1), lambda i: (i, 0)),
                  pl.BlockSpec((_TILE, 1), lambda i: (i, 0))],
        out_specs=[pl.BlockSpec((_TILE, _E), lambda i: (i, 0)),
                   pl.BlockSpec((_TILE, 1), lambda i: (i, 0)),
                   pl.BlockSpec((_TILE, 1), lambda i: (i, 0)),
                   pl.BlockSpec((_TILE, 1), lambda i: (i, 0))],
        out_shape=[jax.ShapeDtypeStruct((_T * _B, _E), jnp.float32),
                   jax.ShapeDtypeStruct((_T * _B, 1), jnp.float32),
                   jax.ShapeDtypeStruct((_T * _B, 1), jnp.float32),
                   jax.ShapeDtypeStruct((_T * _B, 1), jnp.float32)],
    )(ksf, w, enc, minidx)


def _c4_body(en_ref, enc_ref, v_ref, em_ref, led_ref):
    en = en_ref[...]
    enc = enc_ref[...]
    change = (enc[1:, :] - enc[:-1, :]) != 0
    ec = jnp.where(change, 0.0, en[1:, :] - en[:-1, :])
    led = jnp.mean(jnp.maximum(ec + (1e-06 / _NE), 0.0))
    mn = jnp.min(enc, axis=0)
    mx = jnp.max(enc, axis=0)
    v_ref[...] = jnp.reshape(jnp.max(mx - mn), (1, 1))
    em_ref[...] = jnp.reshape(jnp.mean(en), (1, 1))
    led_ref[...] = jnp.reshape(led, (1, 1))


def _call4(en_tb, enc_tb):
    return pl.pallas_call(
        _c4_body,
        grid=(1,),
        in_specs=[pl.BlockSpec((_T, _B), lambda i: (0, 0)),
                  pl.BlockSpec((_T, _B), lambda i: (0, 0))],
        out_specs=[pl.BlockSpec((1, 1), lambda i: (0, 0)),
                   pl.BlockSpec((1, 1), lambda i: (0, 0)),
                   pl.BlockSpec((1, 1), lambda i: (0, 0))],
        out_shape=[jax.ShapeDtypeStruct((1, 1), jnp.int32),
                   jax.ShapeDtypeStruct((1, 1), jnp.float32),
                   jax.ShapeDtypeStruct((1, 1), jnp.float32)],
    )(en_tb, enc_tb)


def kernel(key_soft, W):
    B, T, E = key_soft.shape
    ksf = key_soft.transpose(1, 0, 2).reshape(T * B, E)
    wt = W.T
    w32, minidx = _call1(ksf, wt)
    w32_bm = w32.reshape(T, B, 32).transpose(1, 0, 2).reshape(B, T * 32)
    enc_lanes = _call2_sc(w32_bm, minidx[:B, 0])
    encoding_indices = enc_lanes.reshape(B, T, 16)[:, :, 0]
    enc = encoding_indices.T.reshape(T * B, 1)
    kh, lh, ln, en = _call3(ksf, W, enc, minidx)
    v, em, led = _call4(en.reshape(T, B), enc.reshape(T, B))
    key_hard = kh.reshape(T, B, E).transpose(1, 0, 2)
    loss_here = lh.reshape(T, B).T
    loss_next = ln.reshape(T, B).T
    return (key_hard, encoding_indices, v[0, 0], loss_here, loss_next,
            em[0, 0], led[0, 0])
